# Initial kernel scaffold; baseline (speedup 1.0000x reference)
import jax
import jax.numpy as jnp
from jax.experimental import pallas as pl


def kernel(z_s, z_i, labels):
    raise NotImplementedError("write your pallas kernel here")


# speedup vs baseline: 2.4020x; 2.4020x over previous
import functools

import jax
import jax.numpy as jnp
from jax import lax
from jax.experimental import pallas as pl
from jax.experimental.pallas import tpu as pltpu
from jax.experimental.pallas import tpu_sc as plsc

_N = 320000
_D = 128
_C = 512
_WEIGHT = 0.3

_NC = 2
_NS = 16
_NW = _NC * _NS
_ROWS_PER_W = _N // _NW
_CHUNK = 40
_NCHUNK = _ROWS_PER_W // _CHUNK


def _make_sc_kernel():
    mesh = plsc.VectorSubcoreMesh(core_axis_name="c", subcore_axis_name="s",
                                  num_cores=_NC, num_subcores=_NS)
    rows_per_tile = _C // _NS

    @functools.partial(
        pl.kernel,
        out_type=[
            jax.ShapeDtypeStruct((_NC, _C, _D), jnp.float32),
            jax.ShapeDtypeStruct((_NC, _C, _D), jnp.float32),
            jax.ShapeDtypeStruct((_NC, _C, 128), jnp.float32),
        ],
        mesh=mesh,
        scratch_types=[
            pltpu.VMEM((_CHUNK, _D), jnp.float32),
            pltpu.VMEM((_CHUNK, _D), jnp.float32),
            pltpu.VMEM((_CHUNK,), jnp.int32),
            pltpu.VMEM((_CHUNK, 128), jnp.float32),
            pltpu.VMEM((rows_per_tile, 128), jnp.float32),
            pltpu.VMEM_SHARED((_C, _D), jnp.float32),
            pltpu.VMEM_SHARED((_C, _D), jnp.float32),
            pltpu.VMEM_SHARED((_C, 128), jnp.float32),
        ],
    )
    def sc_k(zs, zi, lb, out_s, out_i, out_c,
             bs, bi, lblb, ones_b, zb, sh_s, sh_i, sh_c):
        cid = lax.axis_index("c")
        sid = lax.axis_index("s")
        wid = sid * _NC + cid

        zeros16 = jnp.zeros((16,), jnp.float32)
        ones16 = jnp.ones((16,), jnp.float32)

        def fill_ones(it, _):
            r = it // (_D // 16)
            c = it % (_D // 16)
            ones_b[r, pl.ds(c * 16, 16)] = ones16
            return _
        lax.fori_loop(0, _CHUNK * (_D // 16), fill_ones, None)

        def fill_zb(it, _):
            r = it // (_D // 16)
            c = it % (_D // 16)
            zb[r, pl.ds(c * 16, 16)] = zeros16
            return _
        lax.fori_loop(0, rows_per_tile * (_D // 16), fill_zb, None)

        def fill_bs(it, _):
            r = it // (_D // 16)
            c = it % (_D // 16)
            bs[r, pl.ds(c * 16, 16)] = zeros16
            return _
        lax.fori_loop(0, rows_per_tile * (_D // 16), fill_bs, None)

        row0 = sid * rows_per_tile
        pltpu.sync_copy(bs.at[pl.ds(0, rows_per_tile)],
                        sh_s.at[pl.ds(row0, rows_per_tile)])
        pltpu.sync_copy(bs.at[pl.ds(0, rows_per_tile)],
                        sh_i.at[pl.ds(row0, rows_per_tile)])
        pltpu.sync_copy(zb, sh_c.at[pl.ds(row0, rows_per_tile)])
        plsc.subcore_barrier()

        base0 = wid * _ROWS_PER_W

        def step(g, _):
            base = base0 + g * _CHUNK
            pltpu.sync_copy(lb.at[pl.ds(base, _CHUNK)], lblb)
            pltpu.sync_copy(zs.at[pl.ds(base, _CHUNK)], bs)
            pltpu.sync_copy(zi.at[pl.ds(base, _CHUNK)], bi)
            pltpu.sync_copy(bs, sh_s.at[lblb], add=True)
            pltpu.sync_copy(bi, sh_i.at[lblb], add=True)
            pltpu.sync_copy(ones_b, sh_c.at[lblb], add=True)
            return _
        lax.fori_loop(0, _NCHUNK, step, None)

        plsc.subcore_barrier()

        pltpu.sync_copy(sh_s.at[pl.ds(row0, rows_per_tile)],
                        bs.at[pl.ds(0, rows_per_tile)])
        pltpu.sync_copy(bs.at[pl.ds(0, rows_per_tile)],
                        out_s.at[cid, pl.ds(row0, rows_per_tile)])
        pltpu.sync_copy(sh_i.at[pl.ds(row0, rows_per_tile)],
                        bi.at[pl.ds(0, rows_per_tile)])
        pltpu.sync_copy(bi.at[pl.ds(0, rows_per_tile)],
                        out_i.at[cid, pl.ds(row0, rows_per_tile)])
        pltpu.sync_copy(sh_c.at[pl.ds(row0, rows_per_tile)], zb)
        pltpu.sync_copy(zb, out_c.at[cid, pl.ds(row0, rows_per_tile)])

    return sc_k


def _fin_body(s_ref, i_ref, c_ref, o_ref):
    s = s_ref[0] + s_ref[1]
    t = i_ref[0] + i_ref[1]
    cnt3 = c_ref[0] + c_ref[1]
    cnt = cnt3[:, 0:1]
    d = s - t
    sq = jnp.sum(d * d, axis=1, keepdims=True)
    present = cnt > 0.0
    denom = jnp.maximum(cnt, 1.0)
    dist = jnp.sqrt(jnp.where(present, sq, 1.0)) / denom
    loss_sum = jnp.sum(jnp.where(present, dist, 0.0))
    npres = jnp.maximum(jnp.sum(jnp.where(present, 1.0, 0.0)), 1.0)
    o_ref[...] = jnp.full((1, 1), _WEIGHT * (loss_sum / npres), jnp.float32)


def kernel(z_s, z_i, labels):
    sc_k = _make_sc_kernel()
    acc_s, acc_i, acc_c = sc_k(z_s, z_i, labels.astype(jnp.int32))
    out = pl.pallas_call(
        _fin_body,
        out_shape=jax.ShapeDtypeStruct((1, 1), jnp.float32),
    )(acc_s, acc_i, acc_c)
    return out[0, 0]


# speedup vs baseline: 4.1712x; 1.7366x over previous
import functools

import jax
import jax.numpy as jnp
from jax import lax
from jax.experimental import pallas as pl
from jax.experimental.pallas import tpu as pltpu
from jax.experimental.pallas import tpu_sc as plsc

_N = 320000
_D = 128
_C = 512
_WEIGHT = 0.3

_NC = 2
_NS = 16
_NW = _NC * _NS
_ROWS_PER_W = _N // _NW
_CHUNK = 80
_NCHUNK = _ROWS_PER_W // _CHUNK
_NSLOT = 4


def _make_sc_kernel():
    mesh = plsc.VectorSubcoreMesh(core_axis_name="c", subcore_axis_name="s",
                                  num_cores=_NC, num_subcores=_NS)
    rows_per_tile = _C // _NS

    @functools.partial(
        pl.kernel,
        out_type=[
            jax.ShapeDtypeStruct((_NC, _C, _D), jnp.float32),
            jax.ShapeDtypeStruct((_NC, _C, _D), jnp.float32),
            jax.ShapeDtypeStruct((_NC, _C, _D), jnp.float32),
        ],
        mesh=mesh,
        scratch_types=[
            pltpu.VMEM((_NSLOT, _CHUNK, _D), jnp.float32),
            pltpu.VMEM((_NSLOT, _CHUNK, _D), jnp.float32),
            pltpu.VMEM((_NSLOT, _CHUNK), jnp.int32),
            pltpu.VMEM((_CHUNK, _D), jnp.float32),
            pltpu.VMEM((rows_per_tile, _D), jnp.float32),
            pltpu.VMEM_SHARED((_C, _D), jnp.float32),
            pltpu.VMEM_SHARED((_C, _D), jnp.float32),
            pltpu.VMEM_SHARED((_C, _D), jnp.float32),
            pltpu.SemaphoreType.DMA((_NSLOT,)),
            pltpu.SemaphoreType.DMA((_NSLOT,)),
        ],
    )
    def sc_k(zs, zi, lb, out_s, out_i, out_c,
             bs, bi, lbl, ones_b, zb, sh_s, sh_i, sh_c, gsem, ssem):
        cid = lax.axis_index("c")
        sid = lax.axis_index("s")
        wid = sid * _NC + cid
        base0 = wid * _ROWS_PER_W

        def issue_gather(c, b):
            base = base0 + c * _CHUNK
            pltpu.async_copy(lb.at[pl.ds(base, _CHUNK)], lbl.at[b],
                             gsem.at[b])
            pltpu.async_copy(zs.at[pl.ds(base, _CHUNK)], bs.at[b],
                             gsem.at[b])
            pltpu.async_copy(zi.at[pl.ds(base, _CHUNK)], bi.at[b],
                             gsem.at[b])

        def wait_gather(c, b):
            base = base0 + c * _CHUNK
            pltpu.make_async_copy(lb.at[pl.ds(base, _CHUNK)], lbl.at[b],
                                  gsem.at[b]).wait()
            pltpu.make_async_copy(zs.at[pl.ds(base, _CHUNK)], bs.at[b],
                                  gsem.at[b]).wait()
            pltpu.make_async_copy(zi.at[pl.ds(base, _CHUNK)], bi.at[b],
                                  gsem.at[b]).wait()

        def issue_scatter(b):
            pltpu.async_copy(bs.at[b], sh_s.at[lbl.at[b]], ssem.at[b],
                             add=True)
            pltpu.async_copy(bi.at[b], sh_i.at[lbl.at[b]], ssem.at[b],
                             add=True)
            pltpu.async_copy(ones_b, sh_c.at[lbl.at[b]], ssem.at[b],
                             add=True)

        def wait_scatter(b):
            pltpu.make_async_copy(bs.at[b], sh_s.at[lbl.at[b]],
                                  ssem.at[b]).wait()
            pltpu.make_async_copy(bi.at[b], sh_i.at[lbl.at[b]],
                                  ssem.at[b]).wait()
            pltpu.make_async_copy(ones_b, sh_c.at[lbl.at[b]],
                                  ssem.at[b]).wait()

        issue_gather(0, 0)
        issue_gather(1, 1)

        zeros16 = jnp.zeros((16,), jnp.float32)
        ones16 = jnp.ones((16,), jnp.float32)

        def fill_ones(it, _):
            r = it // (_D // 16)
            c = it % (_D // 16)
            ones_b[r, pl.ds(c * 16, 16)] = ones16
            return _
        lax.fori_loop(0, _CHUNK * (_D // 16), fill_ones, None)

        def fill_zb(it, _):
            r = it // (_D // 16)
            c = it % (_D // 16)
            zb[r, pl.ds(c * 16, 16)] = zeros16
            return _
        lax.fori_loop(0, rows_per_tile * (_D // 16), fill_zb, None)

        row0 = sid * rows_per_tile
        pltpu.sync_copy(zb, sh_s.at[pl.ds(row0, rows_per_tile)])
        pltpu.sync_copy(zb, sh_i.at[pl.ds(row0, rows_per_tile)])
        pltpu.sync_copy(zb, sh_c.at[pl.ds(row0, rows_per_tile)])
        plsc.subcore_barrier()

        def group(g, _):
            for b in range(_NSLOT):
                c = g * _NSLOT + b
                wait_gather(c, b)
                issue_scatter(b)
                nb = (b + 2) % _NSLOT
                nc = c + 2

                @pl.when(nc >= _NSLOT)
                def _older():
                    wait_scatter(nb)

                @pl.when(nc < _NCHUNK)
                def _prefetch():
                    issue_gather(nc, nb)
            return _
        lax.fori_loop(0, _NCHUNK // _NSLOT, group, None)

        for c in range(_NCHUNK - _NCHUNK % _NSLOT, _NCHUNK):
            b = c % _NSLOT
            wait_gather(c, b)
            issue_scatter(b)

        for b in (2, 3, 0):
            wait_scatter(b)

        plsc.subcore_barrier()

        pltpu.sync_copy(sh_s.at[pl.ds(row0, rows_per_tile)],
                        bs.at[0, pl.ds(0, rows_per_tile)])
        pltpu.sync_copy(bs.at[0, pl.ds(0, rows_per_tile)],
                        out_s.at[cid, pl.ds(row0, rows_per_tile)])
        pltpu.sync_copy(sh_i.at[pl.ds(row0, rows_per_tile)],
                        bi.at[0, pl.ds(0, rows_per_tile)])
        pltpu.sync_copy(bi.at[0, pl.ds(0, rows_per_tile)],
                        out_i.at[cid, pl.ds(row0, rows_per_tile)])
        pltpu.sync_copy(sh_c.at[pl.ds(row0, rows_per_tile)],
                        bs.at[1, pl.ds(0, rows_per_tile)])
        pltpu.sync_copy(bs.at[1, pl.ds(0, rows_per_tile)],
                        out_c.at[cid, pl.ds(row0, rows_per_tile)])

    return sc_k


def _fin_body(s_ref, i_ref, c_ref, o_ref):
    s = s_ref[0] + s_ref[1]
    t = i_ref[0] + i_ref[1]
    cnt3 = c_ref[0] + c_ref[1]
    cnt = cnt3[:, 0:1]
    d = s - t
    sq = jnp.sum(d * d, axis=1, keepdims=True)
    present = cnt > 0.0
    denom = jnp.maximum(cnt, 1.0)
    dist = jnp.sqrt(jnp.where(present, sq, 1.0)) / denom
    loss_sum = jnp.sum(jnp.where(present, dist, 0.0))
    npres = jnp.maximum(jnp.sum(jnp.where(present, 1.0, 0.0)), 1.0)
    o_ref[...] = jnp.full((1, 1), _WEIGHT * (loss_sum / npres), jnp.float32)


def kernel(z_s, z_i, labels):
    sc_k = _make_sc_kernel()
    acc_s, acc_i, acc_c = sc_k(z_s, z_i, labels.astype(jnp.int32))
    out = pl.pallas_call(
        _fin_body,
        out_shape=jax.ShapeDtypeStruct((1, 1), jnp.float32),
    )(acc_s, acc_i, acc_c)
    return out[0, 0]


# speedup vs baseline: 5.3580x; 1.2845x over previous
import functools

import jax
import jax.numpy as jnp
from jax import lax
from jax.experimental import pallas as pl
from jax.experimental.pallas import tpu as pltpu
from jax.experimental.pallas import tpu_sc as plsc

_N = 320000
_D = 128
_C = 512
_WEIGHT = 0.3

_NC = 2
_NS = 16
_NW = _NC * _NS
_ROWS_PER_W = _N // _NW
_CHUNK = 80
_NCHUNK = _ROWS_PER_W // _CHUNK
_NSLOT = 4
_SENT = _ROWS_PER_W


def _make_sc_kernel():
    mesh = plsc.VectorSubcoreMesh(core_axis_name="c", subcore_axis_name="s",
                                  num_cores=_NC, num_subcores=_NS)
    rows_per_tile = _C // _NS

    @functools.partial(
        pl.kernel,
        out_type=[
            jax.ShapeDtypeStruct((_NC, _C, _D), jnp.float32),
            jax.ShapeDtypeStruct((_NC, _C, _D), jnp.float32),
            jax.ShapeDtypeStruct((_NW, _C), jnp.int32),
        ],
        mesh=mesh,
        compiler_params=pltpu.CompilerParams(needs_layout_passes=False),
        scratch_types=[
            pltpu.VMEM((_NSLOT, _CHUNK, _D), jnp.float32),
            pltpu.VMEM((_NSLOT, _CHUNK, _D), jnp.float32),
            pltpu.VMEM((_NSLOT, _CHUNK), jnp.int32),
            pltpu.VMEM((_C,), jnp.int32),
            pltpu.VMEM((rows_per_tile, _D), jnp.float32),
            pltpu.VMEM_SHARED((_C, _D), jnp.float32),
            pltpu.VMEM_SHARED((_C, _D), jnp.float32),
            pltpu.SemaphoreType.DMA((_NSLOT,)),
            pltpu.SemaphoreType.DMA((_NSLOT,)),
        ],
    )
    def sc_k(zs, zi, lb, out_s, out_i, out_fp,
             bs, bi, lbl, fp, zb, sh_s, sh_i, gsem, ssem):
        cid = lax.axis_index("c")
        sid = lax.axis_index("s")
        wid = sid * _NC + cid
        base0 = wid * _ROWS_PER_W

        def issue_gather(c, b):
            base = base0 + c * _CHUNK
            pltpu.async_copy(lb.at[pl.ds(base, _CHUNK)], lbl.at[b],
                             gsem.at[b])
            pltpu.async_copy(zs.at[pl.ds(base, _CHUNK)], bs.at[b],
                             gsem.at[b])
            pltpu.async_copy(zi.at[pl.ds(base, _CHUNK)], bi.at[b],
                             gsem.at[b])

        def wait_gather(c, b):
            base = base0 + c * _CHUNK
            pltpu.make_async_copy(lb.at[pl.ds(base, _CHUNK)], lbl.at[b],
                                  gsem.at[b]).wait()
            pltpu.make_async_copy(zs.at[pl.ds(base, _CHUNK)], bs.at[b],
                                  gsem.at[b]).wait()
            pltpu.make_async_copy(zi.at[pl.ds(base, _CHUNK)], bi.at[b],
                                  gsem.at[b]).wait()

        def issue_scatter(b):
            pltpu.async_copy(bs.at[b], sh_s.at[lbl.at[b]], ssem.at[b],
                             add=True)
            pltpu.async_copy(bi.at[b], sh_i.at[lbl.at[b]], ssem.at[b],
                             add=True)

        def wait_scatter(b):
            pltpu.make_async_copy(bs.at[b], sh_s.at[lbl.at[b]],
                                  ssem.at[b]).wait()
            pltpu.make_async_copy(bi.at[b], sh_i.at[lbl.at[b]],
                                  ssem.at[b]).wait()

        issue_gather(0, 0)
        issue_gather(1, 1)

        zeros16 = jnp.zeros((16,), jnp.float32)
        iota16 = lax.iota(jnp.int32, 16)
        sent16 = jnp.full((16,), _SENT, jnp.int32)
        prev_perm = jnp.maximum(iota16 - 1, 0)

        def fill_zb(it, _):
            r = it // (_D // 16)
            c = it % (_D // 16)
            zb[r, pl.ds(c * 16, 16)] = zeros16
            return _
        lax.fori_loop(0, rows_per_tile * (_D // 16), fill_zb, None)

        def fill_fp(k, _):
            fp[pl.ds(k * 16, 16)] = sent16
            return _
        lax.fori_loop(0, _C // 16, fill_fp, None)

        row0 = sid * rows_per_tile
        pltpu.sync_copy(zb, sh_s.at[pl.ds(row0, rows_per_tile)])
        pltpu.sync_copy(zb, sh_i.at[pl.ds(row0, rows_per_tile)])
        plsc.subcore_barrier()

        def mark_first(c, b, prev_last):
            def vec(k, _):
                x = lbl[b, pl.ds(k * 16, 16)]
                pv = lbl[b, pl.ds(jnp.maximum((k - 1) * 16, 0), 16)]
                carry = jnp.where(k == 0, prev_last, pv[15])
                prev = x.at[prev_perm].get(mode="promise_in_bounds")
                prev = jnp.where(iota16 == 0, carry, prev)
                is_first = x != prev
                pos = (c * _CHUNK + k * 16) + iota16
                plsc.store_scatter(fp, [x], pos, mask=is_first)
                return _
            lax.fori_loop(0, _CHUNK // 16, vec, None)
            return lbl[b, pl.ds(_CHUNK - 16, 16)][15]

        def group(g, prev_last):
            for b in range(_NSLOT):
                c = g * _NSLOT + b
                wait_gather(c, b)
                issue_scatter(b)
                prev_last = mark_first(c, b, prev_last)
                nb = (b + 2) % _NSLOT
                nc = c + 2

                @pl.when(nc >= _NSLOT)
                def _older():
                    wait_scatter(nb)

                @pl.when(nc < _NCHUNK)
                def _prefetch():
                    issue_gather(nc, nb)
            return prev_last
        prev_last = lax.fori_loop(0, _NCHUNK // _NSLOT, group,
                                  jnp.int32(-1))

        for c in range(_NCHUNK - _NCHUNK % _NSLOT, _NCHUNK):
            b = c % _NSLOT
            wait_gather(c, b)
            issue_scatter(b)
            prev_last = mark_first(c, b, prev_last)

        for b in (2, 3, 0):
            wait_scatter(b)

        pltpu.sync_copy(fp, out_fp.at[wid])

        plsc.subcore_barrier()

        pltpu.sync_copy(sh_s.at[pl.ds(row0, rows_per_tile)],
                        bs.at[0, pl.ds(0, rows_per_tile)])
        pltpu.sync_copy(bs.at[0, pl.ds(0, rows_per_tile)],
                        out_s.at[cid, pl.ds(row0, rows_per_tile)])
        pltpu.sync_copy(sh_i.at[pl.ds(row0, rows_per_tile)],
                        bi.at[0, pl.ds(0, rows_per_tile)])
        pltpu.sync_copy(bi.at[0, pl.ds(0, rows_per_tile)],
                        out_i.at[cid, pl.ds(row0, rows_per_tile)])

    return sc_k


def _fin_body(s_ref, i_ref, fp_ref, o_ref):
    s = s_ref[0] + s_ref[1]
    t = i_ref[0] + i_ref[1]
    d = s - t
    sq = jnp.sum(d * d, axis=1, keepdims=True)

    fp = fp_ref[...]
    sent = jnp.int32(_ROWS_PER_W)
    nxt = jnp.concatenate(
        [fp[:, 1:], jnp.full((_NW, 1), sent, jnp.int32)], axis=1)
    sh = 1
    while sh < _C:
        pad = jnp.full((_NW, sh), sent, jnp.int32)
        nxt = jnp.minimum(nxt, jnp.concatenate([nxt[:, sh:], pad], axis=1))
        sh *= 2
    cnt_w = jnp.where(fp < sent, nxt - fp, 0)
    cnt = jnp.sum(cnt_w.astype(jnp.float32), axis=0)[:, None]

    present = cnt > 0.0
    denom = jnp.maximum(cnt, 1.0)
    dist = jnp.sqrt(jnp.where(present, sq, 1.0)) / denom
    loss_sum = jnp.sum(jnp.where(present, dist, 0.0))
    npres = jnp.maximum(jnp.sum(jnp.where(present, 1.0, 0.0)), 1.0)
    o_ref[...] = jnp.full((1, 1), _WEIGHT * (loss_sum / npres), jnp.float32)


def kernel(z_s, z_i, labels):
    sc_k = _make_sc_kernel()
    acc_s, acc_i, fp = sc_k(z_s, z_i, labels.astype(jnp.int32))
    out = pl.pallas_call(
        _fin_body,
        out_shape=jax.ShapeDtypeStruct((1, 1), jnp.float32),
    )(acc_s, acc_i, fp)
    return out[0, 0]


# speedup vs baseline: 7.7151x; 1.4399x over previous
import functools

import jax
import jax.numpy as jnp
from jax import lax
from jax.experimental import pallas as pl
from jax.experimental.pallas import tpu as pltpu
from jax.experimental.pallas import tpu_sc as plsc

_N = 320000
_D = 128
_C = 512
_WEIGHT = 0.3

_NC = 2
_NS = 16
_NW = _NC * _NS
_ROWS_PER_W = _N // _NW
_CHUNK = 80
_NCHUNK = _ROWS_PER_W // _CHUNK
_NSLOT = 4
_SENT = _ROWS_PER_W


def _make_sc_kernel():
    mesh = plsc.VectorSubcoreMesh(core_axis_name="c", subcore_axis_name="s",
                                  num_cores=_NC, num_subcores=_NS)
    rows_per_tile = _C // _NS

    @functools.partial(
        pl.kernel,
        out_type=[
            jax.ShapeDtypeStruct((_NC, _C, _D), jnp.float32),
            jax.ShapeDtypeStruct((_NW, _C), jnp.int32),
        ],
        mesh=mesh,
        compiler_params=pltpu.CompilerParams(needs_layout_passes=False),
        scratch_types=[
            pltpu.VMEM((_NSLOT, _CHUNK, _D), jnp.float32),
            pltpu.VMEM((_NSLOT, _CHUNK, _D), jnp.float32),
            pltpu.VMEM((_NSLOT, _CHUNK), jnp.int32),
            pltpu.VMEM((_C,), jnp.int32),
            pltpu.VMEM((rows_per_tile, _D), jnp.float32),
            pltpu.VMEM_SHARED((_C, _D), jnp.float32),
            pltpu.SemaphoreType.DMA((_NSLOT,)),
            pltpu.SemaphoreType.DMA((_NSLOT,)),
        ],
    )
    def sc_k(zs, zi, lb, out_d, out_fp,
             bs, bi, lbl, fp, zb, sh_d, gsem, ssem):
        cid = lax.axis_index("c")
        sid = lax.axis_index("s")
        wid = sid * _NC + cid
        base0 = wid * _ROWS_PER_W

        def issue_gather(c, b):
            base = base0 + c * _CHUNK
            pltpu.async_copy(lb.at[pl.ds(base, _CHUNK)], lbl.at[b],
                             gsem.at[b])
            pltpu.async_copy(zs.at[pl.ds(base, _CHUNK)], bs.at[b],
                             gsem.at[b])
            pltpu.async_copy(zi.at[pl.ds(base, _CHUNK)], bi.at[b],
                             gsem.at[b])

        def wait_gather(c, b):
            base = base0 + c * _CHUNK
            pltpu.make_async_copy(lb.at[pl.ds(base, _CHUNK)], lbl.at[b],
                                  gsem.at[b]).wait()
            pltpu.make_async_copy(zs.at[pl.ds(base, _CHUNK)], bs.at[b],
                                  gsem.at[b]).wait()
            pltpu.make_async_copy(zi.at[pl.ds(base, _CHUNK)], bi.at[b],
                                  gsem.at[b]).wait()

        def issue_scatter(b):
            pltpu.async_copy(bs.at[b], sh_d.at[lbl.at[b]], ssem.at[b],
                             add=True)

        def wait_scatter(b):
            pltpu.make_async_copy(bs.at[b], sh_d.at[lbl.at[b]],
                                  ssem.at[b]).wait()

        def sub_pass(b):
            def rows(r2, _):
                r = r2 * 2
                for rr in (0, 1):
                    for c8 in range(_D // 16):
                        sl = pl.ds(c8 * 16, 16)
                        x = bs[b, r + rr, sl]
                        y = bi[b, r + rr, sl]
                        bs[b, r + rr, sl] = x - y
                return _
            lax.fori_loop(0, _CHUNK // 2, rows, None)

        issue_gather(0, 0)
        issue_gather(1, 1)

        zeros16 = jnp.zeros((16,), jnp.float32)
        iota16 = lax.iota(jnp.int32, 16)
        sent16 = jnp.full((16,), _SENT, jnp.int32)
        prev_perm = jnp.maximum(iota16 - 1, 0)

        def fill_zb(it, _):
            r = it // (_D // 16)
            c = it % (_D // 16)
            zb[r, pl.ds(c * 16, 16)] = zeros16
            return _
        lax.fori_loop(0, rows_per_tile * (_D // 16), fill_zb, None)

        def fill_fp(k, _):
            fp[pl.ds(k * 16, 16)] = sent16
            return _
        lax.fori_loop(0, _C // 16, fill_fp, None)

        row0 = sid * rows_per_tile
        pltpu.sync_copy(zb, sh_d.at[pl.ds(row0, rows_per_tile)])
        plsc.subcore_barrier()

        def mark_first(c, b, prev_last):
            def vec(k, _):
                x = lbl[b, pl.ds(k * 16, 16)]
                pv = lbl[b, pl.ds(jnp.maximum((k - 1) * 16, 0), 16)]
                carry = jnp.where(k == 0, prev_last, pv[15])
                prev = x.at[prev_perm].get(mode="promise_in_bounds")
                prev = jnp.where(iota16 == 0, carry, prev)
                is_first = x != prev
                pos = (c * _CHUNK + k * 16) + iota16
                plsc.store_scatter(fp, [x], pos, mask=is_first)
                return _
            lax.fori_loop(0, _CHUNK // 16, vec, None)
            return lbl[b, pl.ds(_CHUNK - 16, 16)][15]

        def group(g, prev_last):
            for b in range(_NSLOT):
                c = g * _NSLOT + b
                wait_gather(c, b)
                sub_pass(b)
                issue_scatter(b)
                prev_last = mark_first(c, b, prev_last)
                nb = (b + 2) % _NSLOT
                nc = c + 2

                @pl.when(nc >= _NSLOT)
                def _older():
                    wait_scatter(nb)

                @pl.when(nc < _NCHUNK)
                def _prefetch():
                    issue_gather(nc, nb)
            return prev_last
        prev_last = lax.fori_loop(0, _NCHUNK // _NSLOT, group,
                                  jnp.int32(-1))

        for c in range(_NCHUNK - _NCHUNK % _NSLOT, _NCHUNK):
            b = c % _NSLOT
            wait_gather(c, b)
            sub_pass(b)
            issue_scatter(b)
            prev_last = mark_first(c, b, prev_last)

        for b in (2, 3, 0):
            wait_scatter(b)

        pltpu.sync_copy(fp, out_fp.at[wid])

        plsc.subcore_barrier()

        pltpu.sync_copy(sh_d.at[pl.ds(row0, rows_per_tile)],
                        bs.at[0, pl.ds(0, rows_per_tile)])
        pltpu.sync_copy(bs.at[0, pl.ds(0, rows_per_tile)],
                        out_d.at[cid, pl.ds(row0, rows_per_tile)])

    return sc_k


def _fin_body(d_ref, fp_ref, o_ref):
    d = d_ref[0] + d_ref[1]
    sq = jnp.sum(d * d, axis=1, keepdims=True)

    fp = fp_ref[...]
    sent = jnp.int32(_ROWS_PER_W)
    nxt = jnp.concatenate(
        [fp[:, 1:], jnp.full((_NW, 1), sent, jnp.int32)], axis=1)
    sh = 1
    while sh < _C:
        pad = jnp.full((_NW, sh), sent, jnp.int32)
        nxt = jnp.minimum(nxt, jnp.concatenate([nxt[:, sh:], pad], axis=1))
        sh *= 2
    cnt_w = jnp.where(fp < sent, nxt - fp, 0)
    cnt = jnp.sum(cnt_w.astype(jnp.float32), axis=0)[:, None]

    present = cnt > 0.0
    denom = jnp.maximum(cnt, 1.0)
    dist = jnp.sqrt(jnp.where(present, sq, 1.0)) / denom
    loss_sum = jnp.sum(jnp.where(present, dist, 0.0))
    npres = jnp.maximum(jnp.sum(jnp.where(present, 1.0, 0.0)), 1.0)
    o_ref[...] = jnp.full((1, 1), _WEIGHT * (loss_sum / npres), jnp.float32)


def kernel(z_s, z_i, labels):
    sc_k = _make_sc_kernel()
    acc_d, fp = sc_k(z_s, z_i, labels.astype(jnp.int32))
    out = pl.pallas_call(
        _fin_body,
        out_shape=jax.ShapeDtypeStruct((1, 1), jnp.float32),
    )(acc_d, fp)
    return out[0, 0]
